# baseline (speedup 1.0000x reference)
import jax
import jax.numpy as jnp
from jax import lax
from jax.experimental import pallas as pl
from jax.experimental.pallas import tpu as pltpu

NCLS = 35
KP = 64
LAM = 0.2
D = 512
N = 16384
R = 4096
G = N // R


def _tc_body(x_ref, lab_ref, sums_ref, aux_ref):
    i = pl.program_id(0)

    @pl.when(i == 0)
    def _init():
        sums_ref[...] = jnp.zeros_like(sums_ref)
        aux_ref[...] = jnp.zeros_like(aux_ref)

    x = x_ref[...]
    r = jnp.sum(x * x, axis=1, keepdims=True)
    scale = 1.0 / jnp.maximum(jnp.sqrt(r), 1e-12)
    lab = lab_ref[0, 0, :]
    iota = lax.broadcasted_iota(jnp.int32, (R, KP), 1)
    onehot = (lab[:, None] == iota).astype(jnp.float32)
    oh_scaled = (onehot * scale).astype(jnp.bfloat16)
    sums_ref[...] += lax.dot_general(
        oh_scaled, x.astype(jnp.bfloat16), (((0,), (0,)), ((), ())),
        preferred_element_type=jnp.float32)
    iota2 = lax.broadcasted_iota(jnp.int32, (R, 128), 1)
    v2 = jnp.where(iota2 == 0, 1.0, jnp.where(iota2 == 1, r * scale * scale, 0.0))
    aux_ref[...] += lax.dot_general(
        onehot, v2, (((0,), (0,)), ((), ())),
        preferred_element_type=jnp.float32)


def _fin_body(tsums_ref, taux_ref, out_ref):
    sums = tsums_ref[0:NCLS, :]
    cnt = taux_ref[0:NCLS, 0:1]
    s2 = taux_ref[0:NCLS, 1:2]
    ssq = jnp.sum(sums * sums, axis=1, keepdims=True)
    safe = jnp.maximum(cnt, 1.0)
    contrib = jnp.where(cnt > 0.0, (s2 - ssq / safe) / safe, 0.0)
    out_ref[...] = LAM * jnp.sum(contrib, keepdims=True)


@jax.jit
def kernel(input, label):
    lab3 = label.reshape(G, 1, R)
    tsums, taux = pl.pallas_call(
        _tc_body,
        grid=(G,),
        in_specs=[
            pl.BlockSpec((R, D), lambda i: (i, 0)),
            pl.BlockSpec((1, 1, R), lambda i: (i, 0, 0)),
        ],
        out_specs=[
            pl.BlockSpec((KP, D), lambda i: (0, 0)),
            pl.BlockSpec((KP, 128), lambda i: (0, 0)),
        ],
        out_shape=[
            jax.ShapeDtypeStruct((KP, D), jnp.float32),
            jax.ShapeDtypeStruct((KP, 128), jnp.float32),
        ],
    )(input, lab3)
    out = pl.pallas_call(
        _fin_body,
        in_specs=[
            pl.BlockSpec((KP, D), lambda: (0, 0)),
            pl.BlockSpec((KP, 128), lambda: (0, 0)),
        ],
        out_specs=pl.BlockSpec((1, 1), lambda: (0, 0)),
        out_shape=jax.ShapeDtypeStruct((1, 1), jnp.float32),
    )(tsums, taux)
    return out[0, 0]


# speedup vs baseline: 1.0262x; 1.0262x over previous
import functools

import jax
import jax.numpy as jnp
from jax.experimental import pallas as pl
from jax.experimental.pallas import tpu as pltpu

NCLS = 35
KP = 64
LAM = 0.2
D = 512
N = 16384
R = 4096
G = N // R


def _body(x_ref, lab_ref, out_ref, sums_ref, cnt_ref, s2_ref):
    i = pl.program_id(0)

    @pl.when(i == 0)
    def _init():
        sums_ref[...] = jnp.zeros_like(sums_ref)
        cnt_ref[...] = jnp.zeros_like(cnt_ref)
        s2_ref[...] = jnp.zeros_like(s2_ref)

    x = x_ref[...]
    r = jnp.sum(x * x, axis=1, keepdims=True)
    scale = 1.0 / jnp.maximum(jnp.sqrt(r), 1e-12)
    lab = lab_ref[0, 0, :]
    iota = jax.lax.broadcasted_iota(jnp.int32, (R, KP), 1)
    onehot = (lab[:, None] == iota).astype(jnp.float32)
    oh_scaled = (onehot * scale).astype(jnp.bfloat16)
    sums_ref[...] += jax.lax.dot_general(
        x.astype(jnp.bfloat16), oh_scaled, (((0,), (0,)), ((), ())),
        preferred_element_type=jnp.float32)
    cnt_ref[...] += jnp.sum(onehot, axis=0, keepdims=True)
    s2_ref[...] += jnp.sum(onehot * (r * scale * scale), axis=0,
                           keepdims=True)

    @pl.when(i == G - 1)
    def _epilogue():
        sums = sums_ref[...]
        ssq = jnp.sum(sums * sums, axis=0, keepdims=True)
        cnt = cnt_ref[...]
        safe = jnp.maximum(cnt, 1.0)
        contrib = jnp.where(cnt > 0.0,
                            (s2_ref[...] - ssq / safe) / safe,
                            0.0)
        out_ref[...] = LAM * jnp.sum(contrib, keepdims=True)


@jax.jit
def kernel(input, label):
    lab3 = label.reshape(G, 1, R)
    out = pl.pallas_call(
        _body,
        grid=(G,),
        in_specs=[
            pl.BlockSpec((R, D), lambda i: (i, 0)),
            pl.BlockSpec((1, 1, R), lambda i: (i, 0, 0)),
        ],
        out_specs=pl.BlockSpec((1, 1), lambda i: (0, 0)),
        out_shape=jax.ShapeDtypeStruct((1, 1), jnp.float32),
        scratch_shapes=[
            pltpu.VMEM((D, KP), jnp.float32),
            pltpu.VMEM((1, KP), jnp.float32),
            pltpu.VMEM((1, KP), jnp.float32),
        ],
    )(input, lab3)
    return out[0, 0]


# speedup vs baseline: 1.0316x; 1.0052x over previous
import jax
import jax.numpy as jnp
from jax.experimental import pallas as pl
from jax.experimental.pallas import tpu as pltpu

NCLS = 35
KP = 64
LAM = 0.2
D = 512
N = 16384
R = 4096
G = N // R


def _body(x_ref, lab_ref, out_ref, sums_ref, cnt_ref, s2_ref):
    i = pl.program_id(0)

    @pl.when(i == 0)
    def _init():
        sums_ref[...] = jnp.zeros_like(sums_ref)
        cnt_ref[...] = jnp.zeros_like(cnt_ref)
        s2_ref[...] = jnp.zeros_like(s2_ref)

    x = x_ref[...]
    r = jnp.sum(x * x, axis=1, keepdims=True)
    scale = 1.0 / jnp.maximum(jnp.sqrt(r), 1e-12)
    lab = lab_ref[0, 0, :]
    iota = jax.lax.broadcasted_iota(jnp.int32, (R, KP), 1)
    onehot = (lab[:, None] == iota).astype(jnp.float32)
    oh_scaled = (onehot * scale).astype(jnp.bfloat16)
    sums_ref[...] += jax.lax.dot_general(
        x.astype(jnp.bfloat16), oh_scaled, (((0,), (0,)), ((), ())),
        preferred_element_type=jnp.float32)
    cnt_ref[...] += jnp.sum(onehot, axis=0, keepdims=True)
    s2_ref[...] += jnp.sum(onehot * (r * scale * scale), axis=0,
                           keepdims=True)

    @pl.when(i == G - 1)
    def _epilogue():
        sums = sums_ref[...]
        ssq = jnp.sum(sums * sums, axis=0, keepdims=True)
        cnt = cnt_ref[...]
        safe = jnp.maximum(cnt, 1.0)
        contrib = jnp.where(cnt > 0.0,
                            (s2_ref[...] - ssq / safe) / safe,
                            0.0)
        out_ref[...] = LAM * jnp.sum(contrib, keepdims=True)


@jax.jit
def kernel(input, label):
    lab3 = label.reshape(G, 1, R)
    out = pl.pallas_call(
        _body,
        grid=(G,),
        in_specs=[
            pl.BlockSpec((R, D), lambda i: (i, 0)),
            pl.BlockSpec((1, 1, R), lambda i: (i, 0, 0)),
        ],
        out_specs=pl.BlockSpec((1, 1), lambda i: (0, 0)),
        out_shape=jax.ShapeDtypeStruct((1, 1), jnp.float32),
        scratch_shapes=[
            pltpu.VMEM((D, KP), jnp.float32),
            pltpu.VMEM((1, KP), jnp.float32),
            pltpu.VMEM((1, KP), jnp.float32),
        ],
    )(input, lab3)
    return out[0, 0]
